# baseline (speedup 1.0000x reference)
import functools

import jax
import jax.numpy as jnp
from jax import lax
from jax.experimental import pallas as pl
from jax.experimental.pallas import tpu as pltpu
from jax.experimental.pallas import tpu_sc as plsc

N = 10000
E = 320000
D = 128
NCORES, NS, L = 1, 16, 16
NW = NCORES * NS
CH = 128
CHUNKS = 160
HALF = 40
EPAD = NW * CHUNKS * CH
NPAD = 10240
ROWS_PER_SUB = NPAD // NS

_MESH = plsc.VectorSubcoreMesh(
    core_axis_name="c", subcore_axis_name="s", num_cores=NCORES)


@functools.partial(
    pl.kernel,
    out_type=jax.ShapeDtypeStruct((NPAD,), jnp.float32),
    mesh=_MESH,
    scratch_types=[
        pltpu.VMEM((CHUNKS, CH), jnp.int32),
        pltpu.VMEM((CH,), jnp.float32),
        pltpu.VMEM((ROWS_PER_SUB,), jnp.float32),
        pltpu.VMEM_SHARED((NPAD,), jnp.float32),
    ],
)
def _deg_kernel(dst_hbm, deg_hbm, idx_v, ones_v, zer_v, acc_sh):
    s = lax.axis_index("s")
    w = s

    for i in range(CH // L):
        ones_v[pl.ds(i * L, L)] = jnp.ones((L,), jnp.float32)
    for i in range(ROWS_PER_SUB // L):
        zer_v[pl.ds(i * L, L)] = jnp.zeros((L,), jnp.float32)

    pltpu.sync_copy(zer_v, acc_sh.at[pl.ds(s * ROWS_PER_SUB, ROWS_PER_SUB)])
    pltpu.sync_copy(dst_hbm.at[w], idx_v)
    plsc.subcore_barrier()

    def body(j, carry):
        pltpu.sync_copy(ones_v, acc_sh.at[idx_v.at[j]], add=True)
        return carry

    lax.fori_loop(0, CHUNKS, body, 0)
    plsc.subcore_barrier()

    pltpu.sync_copy(
        acc_sh.at[pl.ds(s * ROWS_PER_SUB, ROWS_PER_SUB)],
        deg_hbm.at[pl.ds(s * ROWS_PER_SUB, ROWS_PER_SUB)],
    )


@functools.partial(
    pl.kernel,
    out_type=jax.ShapeDtypeStruct((NPAD, D), jnp.float32),
    mesh=_MESH,
    scratch_types=[
        pltpu.VMEM((HALF, CH), jnp.int32),
        pltpu.VMEM((HALF, CH), jnp.int32),
        pltpu.VMEM((2 * CH, D), jnp.float32),
        pltpu.VMEM_SHARED((NPAD, D), jnp.float32),
        pltpu.SemaphoreType.DMA,
        pltpu.SemaphoreType.DMA,
    ],
)
def _agg_kernel(g_hbm, src_hbm, dst_hbm, out_hbm,
                sidx_v, didx_v, rows_v, acc_sh, sem0, sem1):
    s = lax.axis_index("s")
    w = s

    def zbody(k, carry):
        i = k // (D // L)
        j = k % (D // L)
        rows_v[i, pl.ds(j * L, L)] = jnp.zeros((L,), jnp.float32)
        return carry

    lax.fori_loop(0, 2 * CH * (D // L), zbody, 0)
    base = s * ROWS_PER_SUB
    for i in range(ROWS_PER_SUB // (2 * CH)):
        pltpu.sync_copy(rows_v, acc_sh.at[pl.ds(base + i * 2 * CH, 2 * CH)])
    rem = ROWS_PER_SUB % (2 * CH)
    if rem:
        pltpu.sync_copy(rows_v.at[pl.ds(0, rem)],
                        acc_sh.at[pl.ds(base + ROWS_PER_SUB - rem, rem)])
    plsc.subcore_barrier()

    slot0 = rows_v.at[pl.ds(0, CH)]
    slot1 = rows_v.at[pl.ds(CH, CH)]

    def gstart(j, slot, sem):
        pltpu.async_copy(g_hbm.at[sidx_v.at[j]], slot, sem)

    def gwait(j, slot, sem):
        pltpu.make_async_copy(g_hbm.at[sidx_v.at[j]], slot, sem).wait()

    for h in range(CHUNKS // HALF):
        pltpu.sync_copy(src_hbm.at[w, pl.ds(h * HALF, HALF)], sidx_v)
        pltpu.sync_copy(dst_hbm.at[w, pl.ds(h * HALF, HALF)], didx_v)
        gstart(0, slot0, sem0)

        def body(i, carry):
            j0 = 2 * i
            j1 = j0 + 1
            gstart(j1, slot1, sem1)
            gwait(j0, slot0, sem0)
            pltpu.sync_copy(slot0, acc_sh.at[didx_v.at[j0]], add=True)
            gstart(lax.rem(j0 + 2, HALF), slot0, sem0)
            gwait(j1, slot1, sem1)
            pltpu.sync_copy(slot1, acc_sh.at[didx_v.at[j1]], add=True)
            return carry

        lax.fori_loop(0, HALF // 2, body, 0)
        gwait(0, slot0, sem0)
    plsc.subcore_barrier()

    for i in range(ROWS_PER_SUB // 128):
        pltpu.sync_copy(acc_sh.at[pl.ds(base + i * 128, 128)],
                        out_hbm.at[pl.ds(base + i * 128, 128)])


_BLK = 1024


def _mm_body(deg_ref, x_ref, w_ref, g_ref):
    i = pl.program_id(0)
    deg = deg_ref[pl.ds(i * _BLK, _BLK)]
    dblk = jnp.where(deg > 0, lax.rsqrt(deg), 0.0)
    h = lax.dot_general(x_ref[...], w_ref[...], (((1,), (1,)), ((), ())),
                        preferred_element_type=jnp.float32)
    g_ref[...] = h * dblk[:, None]


def _fin_body(deg_ref, acc_ref, b_ref, o_ref):
    i = pl.program_id(0)
    deg = deg_ref[pl.ds(i * _BLK, _BLK)]
    dblk = jnp.where(deg > 0, lax.rsqrt(deg), 0.0)
    o_ref[...] = jnp.maximum(acc_ref[...] * dblk[:, None] + b_ref[0][None, :],
                             0.0)


def kernel(x, edge_index, W, b):
    src = edge_index[0].astype(jnp.int32)
    dst = edge_index[1].astype(jnp.int32)
    pad = EPAD - E
    src_p = jnp.concatenate(
        [src, jnp.zeros((pad,), jnp.int32)]).reshape(NW, CHUNKS, CH)
    dst_p = jnp.concatenate(
        [dst, jnp.full((pad,), NPAD - 1, jnp.int32)]).reshape(NW, CHUNKS, CH)

    deg = _deg_kernel(dst_p)

    grid = (N + _BLK - 1) // _BLK
    g = pl.pallas_call(
        _mm_body,
        grid=(grid,),
        in_specs=[
            pl.BlockSpec((NPAD,), lambda i: (0,)),
            pl.BlockSpec((_BLK, D), lambda i: (i, 0)),
            pl.BlockSpec((D, D), lambda i: (0, 0)),
        ],
        out_specs=pl.BlockSpec((_BLK, D), lambda i: (i, 0)),
        out_shape=jax.ShapeDtypeStruct((N, D), jnp.float32),
    )(deg, x, W)

    acc = _agg_kernel(g, src_p, dst_p)

    out = pl.pallas_call(
        _fin_body,
        grid=(grid,),
        in_specs=[
            pl.BlockSpec((NPAD,), lambda i: (0,)),
            pl.BlockSpec((_BLK, D), lambda i: (i, 0)),
            pl.BlockSpec((1, D), lambda i: (0, 0)),
        ],
        out_specs=pl.BlockSpec((_BLK, D), lambda i: (i, 0)),
        out_shape=jax.ShapeDtypeStruct((N, D), jnp.float32),
    )(deg, acc, b.reshape(1, D))
    return out


# speedup vs baseline: 1.1815x; 1.1815x over previous
import functools

import jax
import jax.numpy as jnp
from jax import lax
from jax.experimental import pallas as pl
from jax.experimental.pallas import tpu as pltpu
from jax.experimental.pallas import tpu_sc as plsc

N = 10000
E = 320000
D = 128
NC, NS, L = 2, 16, 16
CH = 128
HALF = 40
CA = 120
CB = 40
ROWS_A = NS * CA
TOT_ROWS = NS * (CA + CB)
EPAD = TOT_ROWS * CH
NPAD = 10240
ROWS_PER_SUB = NPAD // NS

_MESH = plsc.VectorSubcoreMesh(core_axis_name="c", subcore_axis_name="s")


def _my_rows(c, s):
    base = jnp.where(c == 0, s * CA, ROWS_A + s * CB)
    pieces = jnp.where(c == 0, CA // HALF, CB // HALF)
    return base, pieces


@functools.partial(
    pl.kernel,
    out_type=jax.ShapeDtypeStruct((NC, NPAD), jnp.float32),
    mesh=_MESH,
    scratch_types=[
        pltpu.VMEM((HALF, CH), jnp.int32),
        pltpu.VMEM((CH,), jnp.float32),
        pltpu.VMEM((ROWS_PER_SUB,), jnp.float32),
        pltpu.VMEM_SHARED((NPAD,), jnp.float32),
    ],
)
def _deg_kernel(dst_hbm, deg_hbm, idx_v, ones_v, zer_v, acc_sh):
    c = lax.axis_index("c")
    s = lax.axis_index("s")
    base, pieces = _my_rows(c, s)

    for i in range(CH // L):
        ones_v[pl.ds(i * L, L)] = jnp.ones((L,), jnp.float32)
    for i in range(ROWS_PER_SUB // L):
        zer_v[pl.ds(i * L, L)] = jnp.zeros((L,), jnp.float32)

    pltpu.sync_copy(zer_v, acc_sh.at[pl.ds(s * ROWS_PER_SUB, ROWS_PER_SUB)])
    plsc.subcore_barrier()

    def piece(h, carry):
        pltpu.sync_copy(dst_hbm.at[pl.ds(base + h * HALF, HALF)], idx_v)

        def body(j, cc):
            pltpu.sync_copy(ones_v, acc_sh.at[idx_v.at[j]], add=True)
            return cc

        return lax.fori_loop(0, HALF, body, carry)

    lax.fori_loop(0, pieces, piece, 0)
    plsc.subcore_barrier()

    pltpu.sync_copy(
        acc_sh.at[pl.ds(s * ROWS_PER_SUB, ROWS_PER_SUB)],
        deg_hbm.at[c, pl.ds(s * ROWS_PER_SUB, ROWS_PER_SUB)],
    )


@functools.partial(
    pl.kernel,
    out_type=jax.ShapeDtypeStruct((NC, NPAD, D), jnp.float32),
    mesh=_MESH,
    scratch_types=[
        pltpu.VMEM((HALF, CH), jnp.int32),
        pltpu.VMEM((HALF, CH), jnp.int32),
        pltpu.VMEM((2 * CH, D), jnp.float32),
        pltpu.VMEM_SHARED((NPAD, D), jnp.float32),
        pltpu.SemaphoreType.DMA,
        pltpu.SemaphoreType.DMA,
    ],
)
def _agg_kernel(g_hbm, src_hbm, dst_hbm, out_hbm,
                sidx_v, didx_v, rows_v, acc_sh, sem0, sem1):
    c = lax.axis_index("c")
    s = lax.axis_index("s")
    base, pieces = _my_rows(c, s)

    def zbody(k, carry):
        i = k // (D // L)
        j = k % (D // L)
        rows_v[i, pl.ds(j * L, L)] = jnp.zeros((L,), jnp.float32)
        return carry

    lax.fori_loop(0, 2 * CH * (D // L), zbody, 0)
    abase = s * ROWS_PER_SUB
    for i in range(ROWS_PER_SUB // (2 * CH)):
        pltpu.sync_copy(rows_v, acc_sh.at[pl.ds(abase + i * 2 * CH, 2 * CH)])
    rem = ROWS_PER_SUB % (2 * CH)
    if rem:
        pltpu.sync_copy(rows_v.at[pl.ds(0, rem)],
                        acc_sh.at[pl.ds(abase + ROWS_PER_SUB - rem, rem)])
    plsc.subcore_barrier()

    slot0 = rows_v.at[pl.ds(0, CH)]
    slot1 = rows_v.at[pl.ds(CH, CH)]

    def gstart(j, slot, sem):
        pltpu.async_copy(g_hbm.at[sidx_v.at[j]], slot, sem)

    def gwait(j, slot, sem):
        pltpu.make_async_copy(g_hbm.at[sidx_v.at[j]], slot, sem).wait()

    def piece(h, carry):
        pltpu.sync_copy(src_hbm.at[pl.ds(base + h * HALF, HALF)], sidx_v)
        pltpu.sync_copy(dst_hbm.at[pl.ds(base + h * HALF, HALF)], didx_v)
        gstart(0, slot0, sem0)

        def body(i, cc):
            j0 = 2 * i
            j1 = j0 + 1
            gstart(j1, slot1, sem1)
            gwait(j0, slot0, sem0)
            pltpu.sync_copy(slot0, acc_sh.at[didx_v.at[j0]], add=True)
            gstart(lax.rem(j0 + 2, HALF), slot0, sem0)
            gwait(j1, slot1, sem1)
            pltpu.sync_copy(slot1, acc_sh.at[didx_v.at[j1]], add=True)
            return cc

        carry = lax.fori_loop(0, HALF // 2, body, carry)
        gwait(0, slot0, sem0)
        return carry

    lax.fori_loop(0, pieces, piece, 0)
    plsc.subcore_barrier()

    for i in range(ROWS_PER_SUB // 128):
        pltpu.sync_copy(acc_sh.at[pl.ds(abase + i * 128, 128)],
                        out_hbm.at[c, pl.ds(abase + i * 128, 128)])


_BLK = 1024


def _mm_body(deg_ref, x_ref, w_ref, g_ref):
    i = pl.program_id(0)
    deg = deg_ref[0, pl.ds(i * _BLK, _BLK)] + deg_ref[1, pl.ds(i * _BLK, _BLK)]
    dblk = jnp.where(deg > 0, lax.rsqrt(deg), 0.0)
    h = lax.dot_general(x_ref[...], w_ref[...], (((1,), (1,)), ((), ())),
                        preferred_element_type=jnp.float32)
    g_ref[...] = h * dblk[:, None]


def _fin_body(deg_ref, acc_ref, b_ref, o_ref):
    i = pl.program_id(0)
    deg = deg_ref[0, pl.ds(i * _BLK, _BLK)] + deg_ref[1, pl.ds(i * _BLK, _BLK)]
    dblk = jnp.where(deg > 0, lax.rsqrt(deg), 0.0)
    a = acc_ref[0] + acc_ref[1]
    o_ref[...] = jnp.maximum(a * dblk[:, None] + b_ref[0][None, :], 0.0)


def kernel(x, edge_index, W, b):
    src = edge_index[0].astype(jnp.int32)
    dst = edge_index[1].astype(jnp.int32)
    pad = EPAD - E
    src_p = jnp.concatenate(
        [src, jnp.zeros((pad,), jnp.int32)]).reshape(TOT_ROWS, CH)
    dst_p = jnp.concatenate(
        [dst, jnp.full((pad,), NPAD - 1, jnp.int32)]).reshape(TOT_ROWS, CH)

    degp = _deg_kernel(dst_p)

    grid = (N + _BLK - 1) // _BLK
    g = pl.pallas_call(
        _mm_body,
        grid=(grid,),
        in_specs=[
            pl.BlockSpec((NC, NPAD), lambda i: (0, 0)),
            pl.BlockSpec((_BLK, D), lambda i: (i, 0)),
            pl.BlockSpec((D, D), lambda i: (0, 0)),
        ],
        out_specs=pl.BlockSpec((_BLK, D), lambda i: (i, 0)),
        out_shape=jax.ShapeDtypeStruct((N, D), jnp.float32),
    )(degp, x, W)

    accp = _agg_kernel(g, src_p, dst_p)

    out = pl.pallas_call(
        _fin_body,
        grid=(grid,),
        in_specs=[
            pl.BlockSpec((NC, NPAD), lambda i: (0, 0)),
            pl.BlockSpec((NC, _BLK, D), lambda i: (0, i, 0)),
            pl.BlockSpec((1, D), lambda i: (0, 0)),
        ],
        out_specs=pl.BlockSpec((_BLK, D), lambda i: (i, 0)),
        out_shape=jax.ShapeDtypeStruct((N, D), jnp.float32),
    )(degp, accp, b.reshape(1, D))
    return out
